# baseline (device time: 242151 ns/iter reference)
import jax
import jax.numpy as jnp
from jax import lax
from jax.experimental import pallas as pl
from jax.experimental.pallas import tpu as pltpu

N_DEV = 8
SQ = 1024
D = 1024
HQ = 8
DH = 128
SCALE = 0.08838834764831843

R_STEPS = 4
L_STEPS = 3

_NEXT = (1, 2, 3, 7, 0, 4, 5, 6)
_PREV = (4, 0, 1, 2, 5, 6, 7, 3)


def _lookup(table, idx):
    r = jnp.int32(table[0])
    for i in range(1, N_DEV):
        r = jnp.where(idx == i, jnp.int32(table[i]), r)
    return r


def _ring_attn_body(q_ref, kv_ref, out_ref,
                    commR_ref, commL_ref, l_ref, acc_ref,
                    sendR, recvR, sendL, recvL, creditR, creditL):
    my = lax.axis_index("i")
    left = _lookup(_PREV, my)
    right = _lookup(_NEXT, my)

    barrier_sem = pltpu.get_barrier_semaphore()
    for nbr in (left, right):
        pl.semaphore_signal(
            barrier_sem, inc=1,
            device_id=(nbr,), device_id_type=pl.DeviceIdType.MESH,
        )
    pl.semaphore_wait(barrier_sem, 2)

    commR_ref[0] = kv_ref[...]
    commL_ref[0] = kv_ref[...]

    def flash(chunk_refs, first):
        def step(h, carry):
            qh = q_ref[h]
            ps = []
            for cref in chunk_refs:
                s = lax.dot_general(
                    qh, cref[h], (((1,), (1,)), ((), ())),
                    preferred_element_type=jnp.float32,
                ) * SCALE
                ps.append(jnp.exp(s))
            l_new = ps[0].sum(axis=1, keepdims=True)
            for p in ps[1:]:
                l_new = l_new + p.sum(axis=1, keepdims=True)
            pv = lax.dot_general(
                ps[0].astype(jnp.bfloat16), chunk_refs[0][HQ + h],
                (((1,), (0,)), ((), ())),
                preferred_element_type=jnp.float32,
            )
            for p, cref in zip(ps[1:], chunk_refs[1:]):
                pv = pv + lax.dot_general(
                    p.astype(jnp.bfloat16), cref[HQ + h],
                    (((1,), (0,)), ((), ())),
                    preferred_element_type=jnp.float32,
                )
            if first:
                l_ref[h] = l_new
                acc_ref[h] = pv
            else:
                l_ref[h] = l_ref[h] + l_new
                acc_ref[h] = acc_ref[h] + pv
            return carry
        lax.fori_loop(0, HQ, step, 0)

    HALF = SQ // 2
    for s in range(4):
        slot = s % 2
        nxt = (s + 1) % 2

        if s >= 2:
            pl.semaphore_wait(creditR, 1)
        if s < 3:
            srcR = commR_ref.at[slot]
            dstR = commR_ref.at[nxt]
        else:
            srcR = commR_ref.at[slot, :, pl.ds(0, HALF)]
            dstR = commR_ref.at[nxt, :, pl.ds(0, HALF)]
        rdmaR = pltpu.make_async_remote_copy(
            src_ref=srcR,
            dst_ref=dstR,
            send_sem=sendR.at[slot],
            recv_sem=recvR.at[nxt],
            device_id=(right,),
            device_id_type=pl.DeviceIdType.MESH,
        )
        rdmaR.start()
        if s >= 2:
            pl.semaphore_wait(creditL, 1)
        if s < 3:
            srcL = commL_ref.at[slot]
            dstL = commL_ref.at[nxt]
        else:
            srcL = commL_ref.at[slot, :, pl.ds(HALF, HALF)]
            dstL = commL_ref.at[nxt, :, pl.ds(HALF, HALF)]
        rdmaL = pltpu.make_async_remote_copy(
            src_ref=srcL,
            dst_ref=dstL,
            send_sem=sendL.at[slot],
            recv_sem=recvL.at[nxt],
            device_id=(left,),
            device_id_type=pl.DeviceIdType.MESH,
        )
        rdmaL.start()

        if s == 0:
            flash([kv_ref], first=True)
        else:
            flash([commR_ref.at[slot], commL_ref.at[slot]], first=False)

        rdmaR.wait()
        rdmaL.wait()

        if s in (1, 2):
            pl.semaphore_signal(
                creditR, inc=1,
                device_id=(left,), device_id_type=pl.DeviceIdType.MESH,
            )
            pl.semaphore_signal(
                creditL, inc=1,
                device_id=(right,), device_id_type=pl.DeviceIdType.MESH,
            )

    flash([commR_ref.at[0, :, pl.ds(0, HALF)],
           commL_ref.at[0, :, pl.ds(HALF, HALF)]], first=False)

    for h in range(HQ):
        out_ref[:, h * DH:(h + 1) * DH] = acc_ref[h] / l_ref[h]


def kernel(x, Wq, Wk, Wv, Wo):
    x2 = x[0]
    my = lax.axis_index("i")

    pos = (my * SQ + jnp.arange(SQ)).astype(jnp.float32)[:, None]
    inv = 1.0 / (10000.0 ** (jnp.arange(0, DH, 2, dtype=jnp.float32) / DH))
    ang = pos * inv[None, :]
    cos = jnp.repeat(jnp.cos(ang), 2, axis=-1)
    sin = jnp.repeat(jnp.sin(ang), 2, axis=-1)

    def rope(t3):
        t2 = t3.reshape(SQ, HQ, DH // 2, 2)
        tr = jnp.stack([-t2[..., 1], t2[..., 0]], axis=-1).reshape(SQ, HQ, DH)
        return t3 * cos[:, None, :] + tr * sin[:, None, :]

    def heads(t):
        return t.reshape(SQ, HQ, DH).transpose(1, 0, 2)

    def mm(a, b):
        return lax.dot_general(
            a.astype(jnp.bfloat16), b.astype(jnp.bfloat16),
            (((1,), (0,)), ((), ())),
            preferred_element_type=jnp.float32,
        )

    q = heads(rope(mm(x2, Wq).reshape(SQ, HQ, DH)).reshape(SQ, D))
    k = heads(rope(mm(x2, Wk).reshape(SQ, HQ, DH)).reshape(SQ, D))
    v = heads(mm(x2, Wv))
    q = q.astype(jnp.bfloat16)
    kv = jnp.concatenate([k, v], axis=0).astype(jnp.bfloat16)

    ctx = pl.pallas_call(
        _ring_attn_body,
        out_shape=jax.ShapeDtypeStruct((SQ, D), jnp.float32),
        in_specs=[
            pl.BlockSpec(memory_space=pltpu.VMEM),
            pl.BlockSpec(memory_space=pltpu.VMEM),
        ],
        out_specs=pl.BlockSpec(memory_space=pltpu.VMEM),
        scratch_shapes=[
            pltpu.VMEM((2, 2 * HQ, SQ, DH), jnp.bfloat16),
            pltpu.VMEM((2, 2 * HQ, SQ, DH), jnp.bfloat16),
            pltpu.VMEM((HQ, SQ, 1), jnp.float32),
            pltpu.VMEM((HQ, SQ, DH), jnp.float32),
            pltpu.SemaphoreType.DMA((2,)),
            pltpu.SemaphoreType.DMA((2,)),
            pltpu.SemaphoreType.DMA((2,)),
            pltpu.SemaphoreType.DMA((2,)),
            pltpu.SemaphoreType.REGULAR,
            pltpu.SemaphoreType.REGULAR,
        ],
        compiler_params=pltpu.CompilerParams(
            collective_id=0,
            vmem_limit_bytes=100 * 1024 * 1024,
        ),
    )(q, kv)

    return mm(ctx, Wo)[None, :, :]


# device time: 241571 ns/iter; 1.0024x vs baseline; 1.0024x over previous
import jax
import jax.numpy as jnp
from jax import lax
from jax.experimental import pallas as pl
from jax.experimental.pallas import tpu as pltpu

N_DEV = 8
SQ = 1024
D = 1024
HQ = 8
DH = 128
SCALE = 0.08838834764831843

R_STEPS = 4
L_STEPS = 3

_NEXT = (1, 2, 3, 7, 0, 4, 5, 6)
_PREV = (4, 0, 1, 2, 5, 6, 7, 3)


def _lookup(table, idx):
    r = jnp.int32(table[0])
    for i in range(1, N_DEV):
        r = jnp.where(idx == i, jnp.int32(table[i]), r)
    return r


def _ring_attn_body(q_ref, kv_ref, out_ref,
                    commR_ref, commL_ref, l_ref, acc_ref,
                    sendR, recvR, sendL, recvL, creditR, creditL):
    my = lax.axis_index("i")
    left = _lookup(_PREV, my)
    right = _lookup(_NEXT, my)

    barrier_sem = pltpu.get_barrier_semaphore()
    for nbr in (left, right):
        pl.semaphore_signal(
            barrier_sem, inc=1,
            device_id=(nbr,), device_id_type=pl.DeviceIdType.MESH,
        )
    pl.semaphore_wait(barrier_sem, 2)

    commR_ref[0] = kv_ref[...]
    commL_ref[0] = kv_ref[...]

    def flash(chunk_refs, first):
        def step(h, carry):
            qh = q_ref[h]
            ps = []
            for cref in chunk_refs:
                s = lax.dot_general(
                    qh, cref[h], (((1,), (1,)), ((), ())),
                    preferred_element_type=jnp.float32,
                )
                ps.append(jnp.exp2(s * (SCALE * 1.4426950408889634)))
            l_new = ps[0].sum(axis=1, keepdims=True)
            for p in ps[1:]:
                l_new = l_new + p.sum(axis=1, keepdims=True)
            pv = lax.dot_general(
                ps[0].astype(jnp.bfloat16), chunk_refs[0][HQ + h],
                (((1,), (0,)), ((), ())),
                preferred_element_type=jnp.float32,
            )
            for p, cref in zip(ps[1:], chunk_refs[1:]):
                pv = pv + lax.dot_general(
                    p.astype(jnp.bfloat16), cref[HQ + h],
                    (((1,), (0,)), ((), ())),
                    preferred_element_type=jnp.float32,
                )
            if first:
                l_ref[h] = l_new
                acc_ref[h] = pv
            else:
                l_ref[h] = l_ref[h] + l_new
                acc_ref[h] = acc_ref[h] + pv
            return carry
        lax.fori_loop(0, HQ, step, 0)

    HALF = SQ // 2
    for s in range(4):
        slot = s % 2
        nxt = (s + 1) % 2

        if s >= 2:
            pl.semaphore_wait(creditR, 1)
        if s < 3:
            srcR = commR_ref.at[slot]
            dstR = commR_ref.at[nxt]
        else:
            srcR = commR_ref.at[slot, :, pl.ds(0, HALF)]
            dstR = commR_ref.at[nxt, :, pl.ds(0, HALF)]
        rdmaR = pltpu.make_async_remote_copy(
            src_ref=srcR,
            dst_ref=dstR,
            send_sem=sendR.at[slot],
            recv_sem=recvR.at[nxt],
            device_id=(right,),
            device_id_type=pl.DeviceIdType.MESH,
        )
        rdmaR.start()
        if s >= 2:
            pl.semaphore_wait(creditL, 1)
        if s < 3:
            srcL = commL_ref.at[slot]
            dstL = commL_ref.at[nxt]
        else:
            srcL = commL_ref.at[slot, :, pl.ds(HALF, HALF)]
            dstL = commL_ref.at[nxt, :, pl.ds(HALF, HALF)]
        rdmaL = pltpu.make_async_remote_copy(
            src_ref=srcL,
            dst_ref=dstL,
            send_sem=sendL.at[slot],
            recv_sem=recvL.at[nxt],
            device_id=(left,),
            device_id_type=pl.DeviceIdType.MESH,
        )
        rdmaL.start()

        if s == 0:
            flash([kv_ref], first=True)
        else:
            flash([commR_ref.at[slot], commL_ref.at[slot]], first=False)

        rdmaR.wait()
        rdmaL.wait()

        if s in (1, 2):
            pl.semaphore_signal(
                creditR, inc=1,
                device_id=(left,), device_id_type=pl.DeviceIdType.MESH,
            )
            pl.semaphore_signal(
                creditL, inc=1,
                device_id=(right,), device_id_type=pl.DeviceIdType.MESH,
            )

    flash([commR_ref.at[0, :, pl.ds(0, HALF)],
           commL_ref.at[0, :, pl.ds(HALF, HALF)]], first=False)

    for h in range(HQ):
        out_ref[:, h * DH:(h + 1) * DH] = acc_ref[h] / l_ref[h]


def kernel(x, Wq, Wk, Wv, Wo):
    x2 = x[0]
    my = lax.axis_index("i")

    pos = (my * SQ + jnp.arange(SQ)).astype(jnp.float32)[:, None]
    inv = 1.0 / (10000.0 ** (jnp.arange(0, DH, 2, dtype=jnp.float32) / DH))
    ang = pos * inv[None, :]
    cos = jnp.repeat(jnp.cos(ang), 2, axis=-1)
    sin = jnp.repeat(jnp.sin(ang), 2, axis=-1)

    def rope(t3):
        t2 = t3.reshape(SQ, HQ, DH // 2, 2)
        tr = jnp.stack([-t2[..., 1], t2[..., 0]], axis=-1).reshape(SQ, HQ, DH)
        return t3 * cos[:, None, :] + tr * sin[:, None, :]

    def heads(t):
        return t.reshape(SQ, HQ, DH).transpose(1, 0, 2)

    def mm(a, b):
        return lax.dot_general(
            a.astype(jnp.bfloat16), b.astype(jnp.bfloat16),
            (((1,), (0,)), ((), ())),
            preferred_element_type=jnp.float32,
        )

    q = heads(rope(mm(x2, Wq).reshape(SQ, HQ, DH)).reshape(SQ, D))
    k = heads(rope(mm(x2, Wk).reshape(SQ, HQ, DH)).reshape(SQ, D))
    v = heads(mm(x2, Wv))
    q = q.astype(jnp.bfloat16)
    kv = jnp.concatenate([k, v], axis=0).astype(jnp.bfloat16)

    ctx = pl.pallas_call(
        _ring_attn_body,
        out_shape=jax.ShapeDtypeStruct((SQ, D), jnp.float32),
        in_specs=[
            pl.BlockSpec(memory_space=pltpu.VMEM),
            pl.BlockSpec(memory_space=pltpu.VMEM),
        ],
        out_specs=pl.BlockSpec(memory_space=pltpu.VMEM),
        scratch_shapes=[
            pltpu.VMEM((2, 2 * HQ, SQ, DH), jnp.bfloat16),
            pltpu.VMEM((2, 2 * HQ, SQ, DH), jnp.bfloat16),
            pltpu.VMEM((HQ, SQ, 1), jnp.float32),
            pltpu.VMEM((HQ, SQ, DH), jnp.float32),
            pltpu.SemaphoreType.DMA((2,)),
            pltpu.SemaphoreType.DMA((2,)),
            pltpu.SemaphoreType.DMA((2,)),
            pltpu.SemaphoreType.DMA((2,)),
            pltpu.SemaphoreType.REGULAR,
            pltpu.SemaphoreType.REGULAR,
        ],
        compiler_params=pltpu.CompilerParams(
            collective_id=0,
            vmem_limit_bytes=100 * 1024 * 1024,
        ),
    )(q, kv)

    return mm(ctx, Wo)[None, :, :]
